# baseline (speedup 1.0000x reference)
import functools
import jax
import jax.numpy as jnp
from jax import lax
from jax.experimental import pallas as pl
from jax.experimental.pallas import tpu as pltpu
from jax.experimental.pallas import tpu_sc as plsc

_N = 10000
_E = 320000
_H = 128
_NC = 2
_NS = 16
_NW = _NC * _NS
_CH = 128
_NPAD = 10240
_RPW = _NPAD // _NS
_EPW = 10240
_EPAD = _EPW * _NW
_CPW = _EPW // _CH
_DEG_CPW = _EPAD // _NS // _CH

_mesh = plsc.VectorSubcoreMesh(core_axis_name="c", subcore_axis_name="s")


@functools.partial(
    pl.kernel,
    out_type=jax.ShapeDtypeStruct((_NC, _NPAD, _H), jnp.float32),
    mesh=_mesh,
    scratch_types=[
        pltpu.VMEM((_CH,), jnp.int32),
        pltpu.VMEM((_CH,), jnp.int32),
        pltpu.VMEM((_CH, _H), jnp.float32),
        pltpu.VMEM_SHARED((_NPAD, _H), jnp.float32),
        pltpu.SemaphoreType.DMA,
        pltpu.SemaphoreType.DMA,
    ],
)
def _degree_kernel(src_hbm, dst_hbm, ones_hbm, zeros_hbm, out_hbm,
                   di0, di1, ones_v, acc_sh, sem0, sem1):
    c = lax.axis_index("c")
    s = lax.axis_index("s")
    pltpu.sync_copy(zeros_hbm.at[pl.ds(s * _RPW, _RPW)],
                    acc_sh.at[pl.ds(s * _RPW, _RPW)])
    pltpu.sync_copy(ones_hbm, ones_v)
    base = s * _DEG_CPW
    plsc.subcore_barrier()

    @pl.when(c == 0)
    def _():
        pltpu.sync_copy(src_hbm.at[base], di0)

    @pl.when(c == 1)
    def _():
        pltpu.sync_copy(dst_hbm.at[base], di0)

    @pl.loop(0, _DEG_CPW // 2)
    def _pair(p):
        j1 = base + 2 * p + 1

        @pl.when(c == 0)
        def _():
            pltpu.async_copy(src_hbm.at[j1], di1, sem1)

        @pl.when(c == 1)
        def _():
            pltpu.async_copy(dst_hbm.at[j1], di1, sem1)

        pltpu.sync_copy(ones_v, acc_sh.at[di0], add=True)

        @pl.when(p < _DEG_CPW // 2 - 1)
        def _():
            j2 = base + 2 * p + 2

            @pl.when(c == 0)
            def _():
                pltpu.async_copy(src_hbm.at[j2], di0, sem0)

            @pl.when(c == 1)
            def _():
                pltpu.async_copy(dst_hbm.at[j2], di0, sem0)

        pltpu.make_async_copy(src_hbm.at[j1], di1, sem1).wait()
        pltpu.sync_copy(ones_v, acc_sh.at[di1], add=True)

        @pl.when(p < _DEG_CPW // 2 - 1)
        def _():
            pltpu.make_async_copy(src_hbm.at[base], di0, sem0).wait()

    plsc.subcore_barrier()
    pltpu.sync_copy(acc_sh.at[pl.ds(s * _RPW, _RPW)],
                    out_hbm.at[c].at[pl.ds(s * _RPW, _RPW)])


_NBUF = 2


@functools.partial(
    pl.kernel,
    out_type=jax.ShapeDtypeStruct((_NC, _NPAD, _H), jnp.float32),
    mesh=_mesh,
    scratch_types=(
        [pltpu.VMEM((_CPW, _CH // 2), jnp.int32)]
        + [pltpu.VMEM((_CH,), jnp.int32)] * _NBUF
        + [pltpu.VMEM((_CH, _H), jnp.float32)] * _NBUF
        + [pltpu.VMEM_SHARED((_NPAD, _H), jnp.float32)]
        + [pltpu.SemaphoreType.DMA] * (2 * _NBUF)
    ),
)
def _scatter_kernel(table_hbm, src_hbm, dst_hbm, zeros_hbm, out_hbm,
                    si_v, *bufs):
    di = bufs[0:_NBUF]
    rows = bufs[_NBUF:2 * _NBUF]
    acc_sh = bufs[2 * _NBUF]
    gsem = bufs[2 * _NBUF + 1:2 * _NBUF + 1 + _NBUF]
    ssem = bufs[2 * _NBUF + 1 + _NBUF:]
    c = lax.axis_index("c")
    s = lax.axis_index("s")
    w = c * _NS + s
    base = w * _CPW
    hh = _CH // 2
    _HCH = _CPW // 2

    def issue_gather(j, b):
        pltpu.async_copy(table_hbm.at[si_v.at[2 * j]],
                         rows[b].at[pl.ds(0, hh)], gsem[b])
        pltpu.async_copy(table_hbm.at[si_v.at[2 * j + 1]],
                         rows[b].at[pl.ds(hh, hh)], gsem[b])

    def wait_gather(b):
        pltpu.make_async_copy(table_hbm.at[si_v.at[0]],
                              rows[b].at[pl.ds(0, hh)], gsem[b]).wait()
        pltpu.make_async_copy(table_hbm.at[si_v.at[0]],
                              rows[b].at[pl.ds(hh, hh)], gsem[b]).wait()

    pltpu.sync_copy(zeros_hbm.at[pl.ds(s * _RPW, _RPW)],
                    acc_sh.at[pl.ds(s * _RPW, _RPW)])
    plsc.subcore_barrier()

    for h in (0, 1):
        hbase = base + h * _HCH
        pltpu.sync_copy(src_hbm.at[pl.ds((2 * w + h) * _CPW, _CPW)], si_v)
        pltpu.sync_copy(dst_hbm.at[hbase], di[0])
        issue_gather(0, 0)

        @pl.loop(0, _HCH // 2)
        def _pair(p):
            j1 = 2 * p + 1
            pltpu.sync_copy(dst_hbm.at[hbase + j1], di[1])
            issue_gather(j1, 1)
            wait_gather(0)
            pltpu.sync_copy(rows[0], acc_sh.at[di[0]], add=True)

            @pl.when(p < _HCH // 2 - 1)
            def _():
                j2 = 2 * p + 2
                pltpu.sync_copy(dst_hbm.at[hbase + j2], di[0])
                issue_gather(j2, 0)

            wait_gather(1)
            pltpu.sync_copy(rows[1], acc_sh.at[di[1]], add=True)

    plsc.subcore_barrier()
    pltpu.sync_copy(acc_sh.at[pl.ds(s * _RPW, _RPW)],
                    out_hbm.at[c].at[pl.ds(s * _RPW, _RPW)])


_RPB = 640


def _t0_body(x_ref, w_ref, dout_ref, o_ref):
    norm = lax.rsqrt(jnp.maximum(dout_ref[...], 1.0))
    o_ref[...] = jnp.dot(
        x_ref[...], w_ref[...], preferred_element_type=jnp.float32) * norm


def _table0(xpad, W0, dout):
    return pl.pallas_call(
        _t0_body,
        grid=(_NPAD // _RPB,),
        in_specs=[
            pl.BlockSpec((_RPB, _H), lambda i: (i, 0)),
            pl.BlockSpec((_H, _H), lambda i: (0, 0)),
            pl.BlockSpec((_RPB, 1), lambda i: (i, 0)),
        ],
        out_specs=pl.BlockSpec((_RPB, _H), lambda i: (i, 0)),
        out_shape=jax.ShapeDtypeStruct((_NPAD, _H), jnp.float32),
    )(xpad, W0, dout)


def _mid_body(a0_ref, a1_ref, din_ref, dout_ref, w_ref, b_ref, o_ref):
    nin = lax.rsqrt(jnp.maximum(din_ref[...], 1.0))
    h = jnp.maximum((a0_ref[...] + a1_ref[...]) * nin + b_ref[...], 0.0)
    nout = lax.rsqrt(jnp.maximum(dout_ref[...], 1.0))
    o_ref[...] = jnp.dot(
        h, w_ref[...], preferred_element_type=jnp.float32) * nout


def _table_mid(a0, a1, din, dout, W, b):
    return pl.pallas_call(
        _mid_body,
        grid=(_NPAD // _RPB,),
        in_specs=[
            pl.BlockSpec((_RPB, _H), lambda i: (i, 0)),
            pl.BlockSpec((_RPB, _H), lambda i: (i, 0)),
            pl.BlockSpec((_RPB, 1), lambda i: (i, 0)),
            pl.BlockSpec((_RPB, 1), lambda i: (i, 0)),
            pl.BlockSpec((_H, _H), lambda i: (0, 0)),
            pl.BlockSpec((1, _H), lambda i: (0, 0)),
        ],
        out_specs=pl.BlockSpec((_RPB, _H), lambda i: (i, 0)),
        out_shape=jax.ShapeDtypeStruct((_NPAD, _H), jnp.float32),
    )(a0, a1, din, dout, W, b)


def _final_body(a0_ref, a1_ref, din_ref, b_ref, wc_ref, bc_ref, o_ref, acc_ref):
    i = pl.program_id(0)

    @pl.when(i == 0)
    def _():
        acc_ref[...] = jnp.zeros_like(acc_ref)

    nin = lax.rsqrt(jnp.maximum(din_ref[...], 1.0))
    h = jnp.maximum((a0_ref[...] + a1_ref[...]) * nin + b_ref[...], 0.0)
    rows = i * _RPB + lax.broadcasted_iota(jnp.int32, (_RPB, 1), 0)
    h = jnp.where(rows < _N, h, 0.0)
    acc_ref[...] += jnp.sum(h, axis=0, keepdims=True)

    @pl.when(i == pl.num_programs(0) - 1)
    def _():
        o_ref[...] = jnp.dot(
            acc_ref[...] * (1.0 / _N), wc_ref[...],
            preferred_element_type=jnp.float32) + bc_ref[...]


def _final(a0, a1, din, b2, Wc, bc):
    return pl.pallas_call(
        _final_body,
        grid=(_NPAD // _RPB,),
        in_specs=[
            pl.BlockSpec((_RPB, _H), lambda i: (i, 0)),
            pl.BlockSpec((_RPB, _H), lambda i: (i, 0)),
            pl.BlockSpec((_RPB, 1), lambda i: (i, 0)),
            pl.BlockSpec((1, _H), lambda i: (0, 0)),
            pl.BlockSpec((_H, 10), lambda i: (0, 0)),
            pl.BlockSpec((1, 10), lambda i: (0, 0)),
        ],
        out_specs=pl.BlockSpec((1, 10), lambda i: (0, 0)),
        out_shape=jax.ShapeDtypeStruct((1, 10), jnp.float32),
        scratch_shapes=[pltpu.VMEM((1, _H), jnp.float32)],
    )(a0, a1, din, b2, Wc, bc)


def kernel(x, edge_index, W0, b0, W1, b1, W2, b2, Wc, bc):
    src = edge_index[0].astype(jnp.int32)
    dst = edge_index[1].astype(jnp.int32)
    pad_idx = (jnp.arange(_EPAD - _E, dtype=jnp.int32) % (_NPAD - _N)) + _N
    srcp = jnp.concatenate([src, pad_idx])
    dstp = jnp.concatenate([dst, pad_idx])
    src_g = srcp.reshape(_NW * 2 * _CPW, _CH // 2)
    dst_g = dstp.reshape(_NW * _CPW, _CH)
    ones = jnp.ones((_CH, _H), jnp.float32)
    zeros = jnp.zeros((_NPAD, _H), jnp.float32)

    xpad = jnp.pad(x, ((0, _NPAD - _N), (0, 0)))

    degs = _degree_kernel(srcp.reshape(_NW * _CPW, _CH), dst_g, ones, zeros)
    dout = degs[0, :, 0:1]
    din = degs[1, :, 0:1]

    b0r = b0.reshape(1, _H)
    b1r = b1.reshape(1, _H)
    b2r = b2.reshape(1, _H)
    bcr = bc.reshape(1, 10)

    t0 = _table0(xpad, W0, dout)
    a = _scatter_kernel(t0, src_g, dst_g, zeros)
    t1 = _table_mid(a[0], a[1], din, dout, W1, b0r)
    a = _scatter_kernel(t1, src_g, dst_g, zeros)
    t2 = _table_mid(a[0], a[1], din, dout, W2, b1r)
    a = _scatter_kernel(t2, src_g, dst_g, zeros)
    return _final(a[0], a[1], din, b2r, Wc, bcr)


# speedup vs baseline: 1.0934x; 1.0934x over previous
import functools
import jax
import jax.numpy as jnp
from jax import lax
from jax.experimental import pallas as pl
from jax.experimental.pallas import tpu as pltpu
from jax.experimental.pallas import tpu_sc as plsc

_N = 10000
_E = 320000
_H = 128
_NC = 2
_NS = 16
_NW = _NC * _NS
_CH = 128
_NPAD = 10240
_RPW = _NPAD // _NS
_EPW = 10240
_EPAD = _EPW * _NW
_CPW = _EPW // _CH
_DEG_CPW = _EPAD // _NS // _CH

_mesh = plsc.VectorSubcoreMesh(core_axis_name="c", subcore_axis_name="s")


@functools.partial(
    pl.kernel,
    out_type=jax.ShapeDtypeStruct((_NC, _NPAD, _H), jnp.float32),
    mesh=_mesh,
    scratch_types=[
        pltpu.VMEM((_CH,), jnp.int32),
        pltpu.VMEM((_CH,), jnp.int32),
        pltpu.VMEM((_CH, _H), jnp.float32),
        pltpu.VMEM_SHARED((_NPAD, _H), jnp.float32),
        pltpu.SemaphoreType.DMA,
        pltpu.SemaphoreType.DMA,
    ],
)
def _degree_kernel(src_hbm, dst_hbm, ones_hbm, zeros_hbm, out_hbm,
                   di0, di1, ones_v, acc_sh, sem0, sem1):
    c = lax.axis_index("c")
    s = lax.axis_index("s")
    pltpu.sync_copy(zeros_hbm.at[pl.ds(s * _RPW, _RPW)],
                    acc_sh.at[pl.ds(s * _RPW, _RPW)])
    pltpu.sync_copy(ones_hbm, ones_v)
    base = s * _DEG_CPW
    plsc.subcore_barrier()

    @pl.when(c == 0)
    def _():
        pltpu.sync_copy(src_hbm.at[base], di0)

    @pl.when(c == 1)
    def _():
        pltpu.sync_copy(dst_hbm.at[base], di0)

    @pl.loop(0, _DEG_CPW // 2)
    def _pair(p):
        j1 = base + 2 * p + 1

        @pl.when(c == 0)
        def _():
            pltpu.async_copy(src_hbm.at[j1], di1, sem1)

        @pl.when(c == 1)
        def _():
            pltpu.async_copy(dst_hbm.at[j1], di1, sem1)

        pltpu.sync_copy(ones_v, acc_sh.at[di0], add=True)

        @pl.when(p < _DEG_CPW // 2 - 1)
        def _():
            j2 = base + 2 * p + 2

            @pl.when(c == 0)
            def _():
                pltpu.async_copy(src_hbm.at[j2], di0, sem0)

            @pl.when(c == 1)
            def _():
                pltpu.async_copy(dst_hbm.at[j2], di0, sem0)

        pltpu.make_async_copy(src_hbm.at[j1], di1, sem1).wait()
        pltpu.sync_copy(ones_v, acc_sh.at[di1], add=True)

        @pl.when(p < _DEG_CPW // 2 - 1)
        def _():
            pltpu.make_async_copy(src_hbm.at[base], di0, sem0).wait()

    plsc.subcore_barrier()
    pltpu.sync_copy(acc_sh.at[pl.ds(s * _RPW, _RPW)],
                    out_hbm.at[c].at[pl.ds(s * _RPW, _RPW)])


_NBUF = 2


@functools.partial(
    pl.kernel,
    out_type=jax.ShapeDtypeStruct((_NC, _NPAD, _H), jnp.float32),
    mesh=_mesh,
    scratch_types=(
        [pltpu.VMEM((_CPW, _CH), jnp.int32)]
        + [pltpu.VMEM((_CH,), jnp.int32)] * _NBUF
        + [pltpu.VMEM((_CH, _H), jnp.float32)] * _NBUF
        + [pltpu.VMEM_SHARED((_NPAD, _H), jnp.float32)]
        + [pltpu.SemaphoreType.DMA] * (2 * _NBUF)
    ),
)
def _scatter_kernel(table_hbm, src_hbm, dst_hbm, zeros_hbm, out_hbm,
                    si_v, *bufs):
    di = bufs[0:_NBUF]
    rows = bufs[_NBUF:2 * _NBUF]
    acc_sh = bufs[2 * _NBUF]
    gsem = bufs[2 * _NBUF + 1:2 * _NBUF + 1 + _NBUF]
    dsem = bufs[2 * _NBUF + 1 + _NBUF:]
    c = lax.axis_index("c")
    s = lax.axis_index("s")
    w = c * _NS + s
    base = w * _CPW

    pltpu.sync_copy(zeros_hbm.at[pl.ds(s * _RPW, _RPW)],
                    acc_sh.at[pl.ds(s * _RPW, _RPW)])
    pltpu.sync_copy(src_hbm.at[w], si_v)
    plsc.subcore_barrier()

    pltpu.async_copy(dst_hbm.at[base], di[0], dsem[0])
    pltpu.async_copy(table_hbm.at[si_v.at[0]], rows[0], gsem[0])

    @pl.loop(0, _CPW // 2)
    def _pair(p):
        j1 = 2 * p + 1
        pltpu.async_copy(dst_hbm.at[base + j1], di[1], dsem[1])
        pltpu.async_copy(table_hbm.at[si_v.at[j1]], rows[1], gsem[1])
        pltpu.make_async_copy(table_hbm.at[si_v.at[0]], rows[0], gsem[0]).wait()
        pltpu.make_async_copy(dst_hbm.at[base], di[0], dsem[0]).wait()
        pltpu.sync_copy(rows[0], acc_sh.at[di[0]], add=True)

        @pl.when(p < _CPW // 2 - 1)
        def _():
            j2 = 2 * p + 2
            pltpu.async_copy(dst_hbm.at[base + j2], di[0], dsem[0])
            pltpu.async_copy(table_hbm.at[si_v.at[j2]], rows[0], gsem[0])

        pltpu.make_async_copy(table_hbm.at[si_v.at[0]], rows[1], gsem[1]).wait()
        pltpu.make_async_copy(dst_hbm.at[base], di[1], dsem[1]).wait()
        pltpu.sync_copy(rows[1], acc_sh.at[di[1]], add=True)

    plsc.subcore_barrier()
    pltpu.sync_copy(acc_sh.at[pl.ds(s * _RPW, _RPW)],
                    out_hbm.at[c].at[pl.ds(s * _RPW, _RPW)])


_RPB = 640


def _t0_body(x_ref, w_ref, dout_ref, o_ref):
    norm = lax.rsqrt(jnp.maximum(dout_ref[...], 1.0))
    o_ref[...] = jnp.dot(
        x_ref[...], w_ref[...], preferred_element_type=jnp.float32) * norm


def _table0(xpad, W0, dout):
    return pl.pallas_call(
        _t0_body,
        grid=(_NPAD // _RPB,),
        in_specs=[
            pl.BlockSpec((_RPB, _H), lambda i: (i, 0)),
            pl.BlockSpec((_H, _H), lambda i: (0, 0)),
            pl.BlockSpec((_RPB, 1), lambda i: (i, 0)),
        ],
        out_specs=pl.BlockSpec((_RPB, _H), lambda i: (i, 0)),
        out_shape=jax.ShapeDtypeStruct((_NPAD, _H), jnp.float32),
    )(xpad, W0, dout)


def _mid_body(a0_ref, a1_ref, din_ref, dout_ref, w_ref, b_ref, o_ref):
    nin = lax.rsqrt(jnp.maximum(din_ref[...], 1.0))
    h = jnp.maximum((a0_ref[...] + a1_ref[...]) * nin + b_ref[...], 0.0)
    nout = lax.rsqrt(jnp.maximum(dout_ref[...], 1.0))
    o_ref[...] = jnp.dot(
        h, w_ref[...], preferred_element_type=jnp.float32) * nout


def _table_mid(a0, a1, din, dout, W, b):
    return pl.pallas_call(
        _mid_body,
        grid=(_NPAD // _RPB,),
        in_specs=[
            pl.BlockSpec((_RPB, _H), lambda i: (i, 0)),
            pl.BlockSpec((_RPB, _H), lambda i: (i, 0)),
            pl.BlockSpec((_RPB, 1), lambda i: (i, 0)),
            pl.BlockSpec((_RPB, 1), lambda i: (i, 0)),
            pl.BlockSpec((_H, _H), lambda i: (0, 0)),
            pl.BlockSpec((1, _H), lambda i: (0, 0)),
        ],
        out_specs=pl.BlockSpec((_RPB, _H), lambda i: (i, 0)),
        out_shape=jax.ShapeDtypeStruct((_NPAD, _H), jnp.float32),
    )(a0, a1, din, dout, W, b)


def _final_body(a0_ref, a1_ref, din_ref, b_ref, wc_ref, bc_ref, o_ref, acc_ref):
    i = pl.program_id(0)

    @pl.when(i == 0)
    def _():
        acc_ref[...] = jnp.zeros_like(acc_ref)

    nin = lax.rsqrt(jnp.maximum(din_ref[...], 1.0))
    h = jnp.maximum((a0_ref[...] + a1_ref[...]) * nin + b_ref[...], 0.0)
    rows = i * _RPB + lax.broadcasted_iota(jnp.int32, (_RPB, 1), 0)
    h = jnp.where(rows < _N, h, 0.0)
    acc_ref[...] += jnp.sum(h, axis=0, keepdims=True)

    @pl.when(i == pl.num_programs(0) - 1)
    def _():
        o_ref[...] = jnp.dot(
            acc_ref[...] * (1.0 / _N), wc_ref[...],
            preferred_element_type=jnp.float32) + bc_ref[...]


def _final(a0, a1, din, b2, Wc, bc):
    return pl.pallas_call(
        _final_body,
        grid=(_NPAD // _RPB,),
        in_specs=[
            pl.BlockSpec((_RPB, _H), lambda i: (i, 0)),
            pl.BlockSpec((_RPB, _H), lambda i: (i, 0)),
            pl.BlockSpec((_RPB, 1), lambda i: (i, 0)),
            pl.BlockSpec((1, _H), lambda i: (0, 0)),
            pl.BlockSpec((_H, 10), lambda i: (0, 0)),
            pl.BlockSpec((1, 10), lambda i: (0, 0)),
        ],
        out_specs=pl.BlockSpec((1, 10), lambda i: (0, 0)),
        out_shape=jax.ShapeDtypeStruct((1, 10), jnp.float32),
        scratch_shapes=[pltpu.VMEM((1, _H), jnp.float32)],
    )(a0, a1, din, b2, Wc, bc)


def kernel(x, edge_index, W0, b0, W1, b1, W2, b2, Wc, bc):
    src = edge_index[0].astype(jnp.int32)
    dst = edge_index[1].astype(jnp.int32)
    pad_idx = (jnp.arange(_EPAD - _E, dtype=jnp.int32) % (_NPAD - _N)) + _N
    srcp = jnp.concatenate([src, pad_idx])
    dstp = jnp.concatenate([dst, pad_idx])
    src_g = srcp.reshape(_NW, _CPW, _CH)
    dst_g = dstp.reshape(_NW * _CPW, _CH)
    ones = jnp.ones((_CH, _H), jnp.float32)
    zeros = jnp.zeros((_NPAD, _H), jnp.float32)

    xpad = jnp.pad(x, ((0, _NPAD - _N), (0, 0)))

    degs = _degree_kernel(srcp.reshape(_NW * _CPW, _CH), dst_g, ones, zeros)
    dout = degs[0, :, 0:1]
    din = degs[1, :, 0:1]

    b0r = b0.reshape(1, _H)
    b1r = b1.reshape(1, _H)
    b2r = b2.reshape(1, _H)
    bcr = bc.reshape(1, 10)

    t0 = _table0(xpad, W0, dout)
    a = _scatter_kernel(t0, src_g, dst_g, zeros)
    t1 = _table_mid(a[0], a[1], din, dout, W1, b0r)
    a = _scatter_kernel(t1, src_g, dst_g, zeros)
    t2 = _table_mid(a[0], a[1], din, dout, W2, b1r)
    a = _scatter_kernel(t2, src_g, dst_g, zeros)
    return _final(a[0], a[1], din, b2r, Wc, bcr)
